# Initial kernel scaffold; baseline (speedup 1.0000x reference)
import jax
import jax.numpy as jnp
from jax.experimental import pallas as pl


def kernel(x, pos, Wq, bq, Wk, bk, Wv, bv, Wg, bg, Wo, bo):
    raise NotImplementedError("write your pallas kernel here")


# speedup vs baseline: 9.1825x; 9.1825x over previous
import functools

import jax
import jax.numpy as jnp
from jax.experimental import pallas as pl

_B, _C_IN, _C_OUT, _N, _K = 4, 128, 128, 2048, 16
_TN = 512


def _pt_layer_kernel(x_ref, wqk_ref, wv_ref, wog_ref, bqk_ref, bv_ref,
                     bog_ref, out_ref):
    xb = x_ref[0]
    s = jnp.dot(wqk_ref[...], xb, preferred_element_type=jnp.float32)
    s = s + bqk_ref[...]
    m = jnp.max(s, axis=0, keepdims=True)
    e = jnp.exp(s - m)
    attn = e / jnp.sum(e, axis=0, keepdims=True)
    v = jnp.dot(wv_ref[...], xb, preferred_element_type=jnp.float32)
    v = v + bv_ref[...]
    xa = (float(_K) * attn) * v
    out = jnp.dot(wog_ref[...], xa, preferred_element_type=jnp.float32)
    out_ref[0] = out + bog_ref[...]


@functools.partial(jax.jit, static_argnames=())
def kernel(x, pos, Wq, bq, Wk, bk, Wv, bv, Wg, bg, Wo, bo):
    del pos
    B, C_in, N = x.shape
    C_out = Wq.shape[0]

    wqk = Wq - Wk
    bqk = (bq - bk)[:, None]
    wog = Wo + Wo @ Wg
    bog = (Wo @ bg + bo)[:, None]
    bv2 = bv[:, None]

    tn = _TN if N % _TN == 0 else N
    grid = (B, N // tn)

    wspec = pl.BlockSpec((C_out, C_in), lambda b, j: (0, 0))
    bspec = pl.BlockSpec((C_out, 1), lambda b, j: (0, 0))

    out = pl.pallas_call(
        _pt_layer_kernel,
        grid=grid,
        in_specs=[
            pl.BlockSpec((1, C_in, tn), lambda b, j: (b, 0, j)),
            wspec, wspec, wspec, bspec, bspec, bspec,
        ],
        out_specs=pl.BlockSpec((1, C_out, tn), lambda b, j: (b, 0, j)),
        out_shape=jax.ShapeDtypeStruct((B, C_out, N), jnp.float32),
    )(x, wqk, Wv, wog, bqk, bv2, bog)
    return out
